# baseline (speedup 1.0000x reference)
import functools

import jax
import jax.numpy as jnp
from jax import lax
from jax.experimental import pallas as pl
from jax.experimental.pallas import tpu as pltpu
from jax.experimental.pallas import tpu_sc as plsc

_N, _C, _H, _W = 8, 19, 512, 512
_HW = _H * _W
_LANES = 128
_ROWS = _HW // _LANES
_TR = 1024
_K = _ROWS // _TR
_CPAD = 24

_NC, _NS, _L = 2, 16, 16
_NW = _NC * _NS
_PX_PER_W = _HW // (_NW // _N)
_P = 2048
_QT_PER_IMG = _NW // _N


def _main_body(x_ref, q_ref, e_ref):
    k = pl.program_id(1)

    @pl.when(k == 0)
    def _init():
        q_ref[...] = jnp.zeros_like(q_ref)

    x = x_ref[0]

    m = x[0]
    for c in range(1, _C):
        m = jnp.maximum(m, x[c])

    s = jnp.zeros_like(m)
    for c in range(_C):
        e = jnp.exp(x[c] - m)
        e_ref[c] = e
        s = s + e
    r2 = 1.0 / (s * s)

    qrows = []
    for c in range(_C):
        e = e_ref[c]
        qrows.append(jnp.sum(e * e * r2, axis=0, keepdims=True))
    zpad = [jnp.zeros((1, _LANES), jnp.float32)] * (_CPAD - _C)
    q_ref[...] = q_ref[...] + jnp.concatenate(qrows + zpad, axis=0)[None]


def _sc_hist_body(x_hbm, out_hbm, slab_v, hist_v, sem):
    wid = lax.axis_index("s") * _NC + lax.axis_index("c")
    img = wid // _QT_PER_IMG
    qt = wid % _QT_PER_IMG
    pix0 = qt * _PX_PER_W

    zero16 = jnp.zeros((_L,), jnp.float32)
    one16 = jnp.ones((_L,), jnp.float32)

    def vec_body(v, hregs):
        base = v * _L
        m = slab_v[0, pl.ds(base, _L)]
        for c in range(1, _C):
            m = jnp.maximum(m, slab_v[c, pl.ds(base, _L)])
        return tuple(
            hregs[c]
            + jnp.where(slab_v[c, pl.ds(base, _L)] == m, one16, zero16)
            for c in range(_C)
        )

    def chunk_body(ch, hregs):
        off = pix0 + ch * _P
        pltpu.sync_copy(x_hbm.at[img, :, pl.ds(off, _P)], slab_v)
        return lax.fori_loop(0, _P // _L, vec_body, hregs)

    hregs = tuple(zero16 for _ in range(_C))
    hregs = lax.fori_loop(0, _PX_PER_W // _P, chunk_body, hregs)

    for c in range(_C):
        hist_v[c, :] = hregs[c]
    for c in range(_C, _CPAD):
        hist_v[c, :] = zero16

    pltpu.sync_copy(hist_v, out_hbm.at[qt, img])


_sc_hist = functools.partial(
    pl.kernel,
    out_type=jax.ShapeDtypeStruct((_QT_PER_IMG, _N, _CPAD, _L), jnp.float32),
    mesh=plsc.VectorSubcoreMesh(core_axis_name="c", subcore_axis_name="s"),
    scratch_types=[
        pltpu.VMEM((_C, _P), jnp.float32),
        pltpu.VMEM((_CPAD, _L), jnp.float32),
        pltpu.SemaphoreType.DMA,
    ],
    compiler_params=pltpu.CompilerParams(needs_layout_passes=False),
)(_sc_hist_body)


def _combine_body(q_ref, sh_ref, o_ref):
    q = jnp.sum(q_ref[...], axis=2)
    h = jnp.sum(sh_ref[0], axis=2)
    for t in range(1, _QT_PER_IMG):
        h = h + jnp.sum(sh_ref[t], axis=2)
    col = jax.lax.broadcasted_iota(jnp.int32, (_N, _CPAD), 1)
    mask = col < _C
    hadj = jnp.where(h == 0.0, 1.0, h)
    total = jnp.sum(jnp.where(mask, hadj, 0.0), axis=1, keepdims=True)
    w = jnp.exp(0.2 * (jnp.log(total) - jnp.log(hadj)))
    loss = -jnp.sum(jnp.where(mask, w * q, 0.0))
    o_ref[0, 0] = loss * (1.0 / (_N * _C * _H * _W))


def kernel(inputs):
    x = inputs.reshape(_N, _C, _ROWS, _LANES)
    schist = _sc_hist(inputs.reshape(_N, _C, _HW))

    ssq = pl.pallas_call(
        _main_body,
        grid=(_N, _K),
        in_specs=[
            pl.BlockSpec((1, _C, _TR, _LANES), lambda n, k: (n, 0, k, 0)),
        ],
        out_specs=pl.BlockSpec((1, _CPAD, _LANES), lambda n, k: (n, 0, 0)),
        out_shape=jax.ShapeDtypeStruct((_N, _CPAD, _LANES), jnp.float32),
        scratch_shapes=[pltpu.VMEM((_C, _TR, _LANES), jnp.float32)],
        compiler_params=pltpu.CompilerParams(
            dimension_semantics=("parallel", "arbitrary"),
        ),
    )(x)

    out = pl.pallas_call(
        _combine_body,
        out_shape=jax.ShapeDtypeStruct((1, 1), jnp.float32),
        out_specs=pl.BlockSpec(memory_space=pltpu.SMEM),
    )(ssq, schist)
    return out[0, 0]


# speedup vs baseline: 1.7933x; 1.7933x over previous
import functools

import jax
import jax.numpy as jnp
from jax import lax
from jax.experimental import pallas as pl
from jax.experimental.pallas import tpu as pltpu
from jax.experimental.pallas import tpu_sc as plsc

_N, _C, _H, _W = 8, 19, 512, 512
_HW = _H * _W
_LANES = 128
_ROWS = _HW // _LANES
_TR = 1024
_K = _ROWS // _TR
_CPAD = 24

_NC, _NS, _L = 2, 16, 16
_NW = _NC * _NS
_IDS_PER_W = (_N * _HW) // _NW
_QT_PER_IMG = _NW // _N


def _main_body(x_ref, q_ref, idx_ref, e_ref):
    k = pl.program_id(1)

    @pl.when(k == 0)
    def _init():
        q_ref[...] = jnp.zeros_like(q_ref)

    x = x_ref[0]

    m = x[0]
    for c in range(1, _C):
        m = jnp.maximum(m, x[c])

    s = jnp.zeros_like(m)
    idx = jnp.zeros(m.shape, jnp.int32)
    for c in range(_C):
        e = jnp.exp(x[c] - m)
        e_ref[c] = e
        s = s + e
        idx = jnp.where(e == 1.0, c, idx)
    r2 = 1.0 / (s * s)
    idx_ref[0] = idx

    qrows = []
    for c in range(_C):
        e = e_ref[c]
        qrows.append(jnp.sum(e * e * r2, axis=0, keepdims=True))
    zpad = [jnp.zeros((1, _LANES), jnp.float32)] * (_CPAD - _C)
    q_ref[...] = q_ref[...] + jnp.concatenate(qrows + zpad, axis=0)[None]


def _sc_hist_body(idx_hbm, out_hbm, idx_v, hist_v, sem):
    wid = lax.axis_index("s") * _NC + lax.axis_index("c")
    img = wid // _QT_PER_IMG
    qt = wid % _QT_PER_IMG

    pltpu.sync_copy(idx_hbm.at[pl.ds(wid * _IDS_PER_W, _IDS_PER_W)], idx_v)

    for c in range(_CPAD):
        hist_v[c, :] = jnp.zeros((_L,), jnp.float32)

    lane = lax.iota(jnp.int32, _L)
    ones = jnp.ones((_L,), jnp.float32)

    def body(i, carry):
        base = i * (4 * _L)
        for j in range(4):
            iv = idx_v[pl.ds(base + j * _L, _L)]
            plsc.addupdate_scatter(hist_v, [iv, lane], ones)
        return carry

    lax.fori_loop(0, _IDS_PER_W // (4 * _L), body, 0)

    pltpu.sync_copy(hist_v, out_hbm.at[qt, img])


_sc_hist = functools.partial(
    pl.kernel,
    out_type=jax.ShapeDtypeStruct((_QT_PER_IMG, _N, _CPAD, _L), jnp.float32),
    mesh=plsc.VectorSubcoreMesh(core_axis_name="c", subcore_axis_name="s"),
    scratch_types=[
        pltpu.VMEM((_IDS_PER_W,), jnp.int32),
        pltpu.VMEM((_CPAD, _L), jnp.float32),
        pltpu.SemaphoreType.DMA,
    ],
    compiler_params=pltpu.CompilerParams(needs_layout_passes=False),
)(_sc_hist_body)


def _combine_body(q_ref, sh_ref, o_ref):
    q = jnp.sum(q_ref[...], axis=2)
    h = jnp.sum(sh_ref[0], axis=2)
    for t in range(1, _QT_PER_IMG):
        h = h + jnp.sum(sh_ref[t], axis=2)
    col = jax.lax.broadcasted_iota(jnp.int32, (_N, _CPAD), 1)
    mask = col < _C
    hadj = jnp.where(h == 0.0, 1.0, h)
    total = jnp.sum(jnp.where(mask, hadj, 0.0), axis=1, keepdims=True)
    w = jnp.exp(0.2 * (jnp.log(total) - jnp.log(hadj)))
    loss = -jnp.sum(jnp.where(mask, w * q, 0.0))
    o_ref[0, 0] = loss * (1.0 / (_N * _C * _H * _W))


def kernel(inputs):
    x = inputs.reshape(_N, _C, _ROWS, _LANES)
    ssq, idx = pl.pallas_call(
        _main_body,
        grid=(_N, _K),
        in_specs=[
            pl.BlockSpec((1, _C, _TR, _LANES), lambda n, k: (n, 0, k, 0)),
        ],
        out_specs=[
            pl.BlockSpec((1, _CPAD, _LANES), lambda n, k: (n, 0, 0)),
            pl.BlockSpec((1, _TR, _LANES), lambda n, k: (n, k, 0)),
        ],
        out_shape=[
            jax.ShapeDtypeStruct((_N, _CPAD, _LANES), jnp.float32),
            jax.ShapeDtypeStruct((_N, _ROWS, _LANES), jnp.int32),
        ],
        scratch_shapes=[pltpu.VMEM((_C, _TR, _LANES), jnp.float32)],
        compiler_params=pltpu.CompilerParams(
            dimension_semantics=("parallel", "arbitrary"),
        ),
    )(x)

    schist = _sc_hist(idx.reshape(-1))

    out = pl.pallas_call(
        _combine_body,
        out_shape=jax.ShapeDtypeStruct((1, 1), jnp.float32),
        out_specs=pl.BlockSpec(memory_space=pltpu.SMEM),
    )(ssq, schist)
    return out[0, 0]


# speedup vs baseline: 1.7981x; 1.0027x over previous
import functools

import jax
import jax.numpy as jnp
from jax import lax
from jax.experimental import pallas as pl
from jax.experimental.pallas import tpu as pltpu
from jax.experimental.pallas import tpu_sc as plsc

_N, _C, _H, _W = 8, 19, 512, 512
_HW = _H * _W
_LANES = 128
_ROWS = _HW // _LANES
_TR = 1024
_K = _ROWS // _TR
_CPAD = 24

_NC, _NS, _L = 2, 16, 16
_NW = _NC * _NS
_IDS_PER_W = (_N * _HW) // _NW
_QT_PER_IMG = _NW // _N


def _main_body(x_ref, q_ref, idx_ref, e_ref):
    k = pl.program_id(1)

    @pl.when(k == 0)
    def _init():
        q_ref[...] = jnp.zeros_like(q_ref)

    x = x_ref[0]

    m = x[0]
    for c in range(1, _C):
        m = jnp.maximum(m, x[c])

    s = jnp.zeros_like(m)
    idx = jnp.zeros(m.shape, jnp.int32)
    for c in range(_C):
        e = jnp.exp(x[c] - m)
        e_ref[c] = e
        s = s + e
        idx = jnp.where(e == 1.0, c, idx)
    r2 = 1.0 / (s * s)
    idx_ref[0] = idx

    qrows = []
    for c in range(_C):
        e = e_ref[c]
        qrows.append(jnp.sum(e * e * r2, axis=0, keepdims=True))
    zpad = [jnp.zeros((1, _LANES), jnp.float32)] * (_CPAD - _C)
    q_ref[...] = q_ref[...] + jnp.concatenate(qrows + zpad, axis=0)[None]


def _sc_hist_body(idx_hbm, out_hbm, idx_v, hist_v, sem):
    wid = lax.axis_index("s") * _NC + lax.axis_index("c")
    img = wid // _QT_PER_IMG
    qt = wid % _QT_PER_IMG

    pltpu.sync_copy(idx_hbm.at[pl.ds(wid * _IDS_PER_W, _IDS_PER_W)], idx_v)

    for c in range(_CPAD):
        hist_v[c, :] = jnp.zeros((_L,), jnp.float32)

    lane = lax.iota(jnp.int32, _L)
    ones = jnp.ones((_L,), jnp.float32)

    def body(i, carry):
        base = i * (8 * _L)
        for j in range(8):
            iv = idx_v[pl.ds(base + j * _L, _L)]
            plsc.addupdate_scatter(hist_v, [iv, lane], ones)
        return carry

    lax.fori_loop(0, _IDS_PER_W // (8 * _L), body, 0)

    pltpu.sync_copy(hist_v, out_hbm.at[qt, img])


_sc_hist = functools.partial(
    pl.kernel,
    out_type=jax.ShapeDtypeStruct((_QT_PER_IMG, _N, _CPAD, _L), jnp.float32),
    mesh=plsc.VectorSubcoreMesh(core_axis_name="c", subcore_axis_name="s"),
    scratch_types=[
        pltpu.VMEM((_IDS_PER_W,), jnp.int32),
        pltpu.VMEM((_CPAD, _L), jnp.float32),
        pltpu.SemaphoreType.DMA,
    ],
    compiler_params=pltpu.CompilerParams(needs_layout_passes=False),
)(_sc_hist_body)


def _combine_body(q_ref, sh_ref, o_ref):
    q = jnp.sum(q_ref[...], axis=2)
    h = jnp.sum(sh_ref[0], axis=2)
    for t in range(1, _QT_PER_IMG):
        h = h + jnp.sum(sh_ref[t], axis=2)
    col = jax.lax.broadcasted_iota(jnp.int32, (_N, _CPAD), 1)
    mask = col < _C
    hadj = jnp.where(h == 0.0, 1.0, h)
    total = jnp.sum(jnp.where(mask, hadj, 0.0), axis=1, keepdims=True)
    w = jnp.exp(0.2 * (jnp.log(total) - jnp.log(hadj)))
    loss = -jnp.sum(jnp.where(mask, w * q, 0.0))
    o_ref[0, 0] = loss * (1.0 / (_N * _C * _H * _W))


def kernel(inputs):
    x = inputs.reshape(_N, _C, _ROWS, _LANES)
    ssq, idx = pl.pallas_call(
        _main_body,
        grid=(_N, _K),
        in_specs=[
            pl.BlockSpec((1, _C, _TR, _LANES), lambda n, k: (n, 0, k, 0)),
        ],
        out_specs=[
            pl.BlockSpec((1, _CPAD, _LANES), lambda n, k: (n, 0, 0)),
            pl.BlockSpec((1, _TR, _LANES), lambda n, k: (n, k, 0)),
        ],
        out_shape=[
            jax.ShapeDtypeStruct((_N, _CPAD, _LANES), jnp.float32),
            jax.ShapeDtypeStruct((_N, _ROWS, _LANES), jnp.int32),
        ],
        scratch_shapes=[pltpu.VMEM((_C, _TR, _LANES), jnp.float32)],
        compiler_params=pltpu.CompilerParams(
            dimension_semantics=("parallel", "arbitrary"),
        ),
    )(x)

    schist = _sc_hist(idx.reshape(-1))

    out = pl.pallas_call(
        _combine_body,
        out_shape=jax.ShapeDtypeStruct((1, 1), jnp.float32),
        out_specs=pl.BlockSpec(memory_space=pltpu.SMEM),
    )(ssq, schist)
    return out[0, 0]
